# Initial kernel scaffold; baseline (speedup 1.0000x reference)
import jax
import jax.numpy as jnp
from jax.experimental import pallas as pl


def kernel(x, edge_index, in_degrees, W_self0, W_neigh0, b0, W_self1, W_neigh1, b1, W_self2, W_neigh2, b2):
    raise NotImplementedError("write your pallas kernel here")


# speedup vs baseline: 7.5602x; 7.5602x over previous
import functools

import jax
import jax.numpy as jnp
from jax import lax
from jax.experimental import pallas as pl
from jax.experimental.pallas import tpu as pltpu
from jax.experimental.pallas import tpu_sc as plsc

_N = 10000
_E = 320000
_D = 128
_NCORES = 2
_NSUB = 16
_NW = _NCORES * _NSUB
_EW = _E // _NW
_G = 80
_KPT = _EW // _G
_ROWS_PER_SUB = _N // _NSUB
_BN = 1000


def _make_sc_agg(dp: int):
  mesh = plsc.VectorSubcoreMesh(core_axis_name="c", subcore_axis_name="s")

  @functools.partial(
      pl.kernel,
      mesh=mesh,
      compiler_params=pltpu.CompilerParams(use_tc_tiling_on_sc=False),
      out_type=jax.ShapeDtypeStruct((_NCORES * _N, dp), jnp.float32),
      scratch_types=[
          pltpu.VMEM((_KPT, _G), jnp.int32),
          pltpu.VMEM((_KPT, _G), jnp.int32),
          pltpu.VMEM((_G, dp), jnp.float32),
          pltpu.VMEM_SHARED((_N, dp), jnp.float32),
          pltpu.SemaphoreType.DMA,
      ],
  )
  def sc_agg(p_hbm, src_hbm, dst_hbm, zeros_hbm, out_hbm,
             src_v, dst_v, rows_v, agg_sh, sem):
    c = lax.axis_index("c")
    s = lax.axis_index("s")
    w = c * _NSUB + s

    zbase = s * _ROWS_PER_SUB
    pltpu.sync_copy(zeros_hbm.at[pl.ds(zbase, _ROWS_PER_SUB)],
                    agg_sh.at[pl.ds(zbase, _ROWS_PER_SUB)])
    pltpu.sync_copy(src_hbm.at[pl.ds(w * _KPT, _KPT)], src_v)
    pltpu.sync_copy(dst_hbm.at[pl.ds(w * _KPT, _KPT)], dst_v)
    plsc.subcore_barrier()

    def body(k, carry):
      pltpu.async_copy(p_hbm.at[src_v.at[k]], rows_v, sem).wait()
      pltpu.sync_copy(rows_v, agg_sh.at[dst_v.at[k]], add=True)
      return carry

    lax.fori_loop(0, _KPT, body, 0)
    plsc.subcore_barrier()

    pltpu.sync_copy(agg_sh.at[pl.ds(zbase, _ROWS_PER_SUB)],
                    out_hbm.at[pl.ds(c * _N + zbase, _ROWS_PER_SUB)])

  return sc_agg


_sc_agg_128 = _make_sc_agg(_D)
_sc_agg_64 = _make_sc_agg(64)


def _tc_first(x, w_self, w_neigh, b):
  def body(x_ref, ws_ref, wn_ref, b_ref, s_ref, p_ref):
    xb = x_ref[...]
    s_ref[...] = jnp.dot(xb, ws_ref[...],
                         preferred_element_type=jnp.float32) + b_ref[...]
    p_ref[...] = jnp.dot(xb, wn_ref[...], preferred_element_type=jnp.float32)

  return pl.pallas_call(
      body,
      grid=(_N // _BN,),
      in_specs=[
          pl.BlockSpec((_BN, _D), lambda i: (i, 0)),
          pl.BlockSpec((_D, _D), lambda i: (0, 0)),
          pl.BlockSpec((_D, _D), lambda i: (0, 0)),
          pl.BlockSpec((1, _D), lambda i: (0, 0)),
      ],
      out_specs=[
          pl.BlockSpec((_BN, _D), lambda i: (i, 0)),
          pl.BlockSpec((_BN, _D), lambda i: (i, 0)),
      ],
      out_shape=[jax.ShapeDtypeStruct((_N, _D), jnp.float32)] * 2,
  )(x, w_self, w_neigh, b.reshape(1, _D))


def _tc_combine_next(s_prev, parts, deg, w_self, w_neigh, b, dp_in, dp_out):
  def body(s_ref, a0_ref, a1_ref, deg_ref, ws_ref, wn_ref, b_ref,
           s_out, p_out):
    h = s_ref[...] + (a0_ref[...] + a1_ref[...]) / deg_ref[...]
    h = jnp.maximum(h, 0.0)
    s_out[...] = jnp.dot(h, ws_ref[...],
                         preferred_element_type=jnp.float32) + b_ref[...]
    p_out[...] = jnp.dot(h, wn_ref[...], preferred_element_type=jnp.float32)

  nblk = _N // _BN
  return pl.pallas_call(
      body,
      grid=(nblk,),
      in_specs=[
          pl.BlockSpec((_BN, dp_in), lambda i: (i, 0)),
          pl.BlockSpec((_BN, dp_in), lambda i: (i, 0)),
          pl.BlockSpec((_BN, dp_in), lambda i: (i + nblk, 0)),
          pl.BlockSpec((_BN, 1), lambda i: (i, 0)),
          pl.BlockSpec((dp_in, dp_out), lambda i: (0, 0)),
          pl.BlockSpec((dp_in, dp_out), lambda i: (0, 0)),
          pl.BlockSpec((1, dp_out), lambda i: (0, 0)),
      ],
      out_specs=[
          pl.BlockSpec((_BN, dp_out), lambda i: (i, 0)),
          pl.BlockSpec((_BN, dp_out), lambda i: (i, 0)),
      ],
      out_shape=[jax.ShapeDtypeStruct((_N, dp_out), jnp.float32)] * 2,
  )(s_prev, parts, parts, deg, w_self, w_neigh, b.reshape(1, dp_out))


def _tc_final(s_prev, parts, deg, dp):
  def body(s_ref, a0_ref, a1_ref, deg_ref, o_ref):
    o_ref[...] = s_ref[...] + (a0_ref[...] + a1_ref[...]) / deg_ref[...]

  nblk = _N // _BN
  return pl.pallas_call(
      body,
      grid=(nblk,),
      in_specs=[
          pl.BlockSpec((_BN, dp), lambda i: (i, 0)),
          pl.BlockSpec((_BN, dp), lambda i: (i, 0)),
          pl.BlockSpec((_BN, dp), lambda i: (i + nblk, 0)),
          pl.BlockSpec((_BN, 1), lambda i: (i, 0)),
      ],
      out_specs=pl.BlockSpec((_BN, dp), lambda i: (i, 0)),
      out_shape=jax.ShapeDtypeStruct((_N, dp), jnp.float32),
  )(s_prev, parts, parts, deg)


def kernel(x, edge_index, in_degrees,
           W_self0, W_neigh0, b0,
           W_self1, W_neigh1, b1,
           W_self2, W_neigh2, b2):
  src = edge_index[0].reshape(_E // _G, _G)
  dst = edge_index[1].reshape(_E // _G, _G)
  deg = in_degrees.reshape(_N, 1)
  zeros128 = jnp.zeros((_N, _D), jnp.float32)
  zeros64 = jnp.zeros((_N, 64), jnp.float32)

  w_self2p = jnp.pad(W_self2, ((0, 0), (0, 64 - 47)))
  w_neigh2p = jnp.pad(W_neigh2, ((0, 0), (0, 64 - 47)))
  b2p = jnp.pad(b2, (0, 64 - 47))

  s0, p0 = _tc_first(x, W_self0, W_neigh0, b0)
  a0 = _sc_agg_128(p0, src, dst, zeros128)
  s1, p1 = _tc_combine_next(s0, a0, deg, W_self1, W_neigh1, b1, _D, _D)
  a1 = _sc_agg_128(p1, src, dst, zeros128)
  s2, p2 = _tc_combine_next(s1, a1, deg, w_self2p, w_neigh2p, b2p, _D, 64)
  a2 = _sc_agg_64(p2, src, dst, zeros64)
  out = _tc_final(s2, a2, deg, 64)
  return out[:, :47]


# speedup vs baseline: 9.2461x; 1.2230x over previous
import functools

import jax
import jax.numpy as jnp
from jax import lax
from jax.experimental import pallas as pl
from jax.experimental.pallas import tpu as pltpu
from jax.experimental.pallas import tpu_sc as plsc

_N = 10000
_E = 320000
_D = 128
_NCORES = 2
_NSUB = 16
_NW = _NCORES * _NSUB
_EW = _E // _NW
_G = 50
_KPT = _EW // _G
_NBUF = 4
_ROWS_PER_SUB = _N // _NSUB
_BN = 1000


def _make_sc_agg(dp: int):
  mesh = plsc.VectorSubcoreMesh(core_axis_name="c", subcore_axis_name="s")

  @functools.partial(
      pl.kernel,
      mesh=mesh,
      compiler_params=pltpu.CompilerParams(use_tc_tiling_on_sc=False),
      out_type=jax.ShapeDtypeStruct((_NCORES * _N, dp), jnp.float32),
      scratch_types=[
          pltpu.VMEM((_KPT, _G), jnp.int32),
          pltpu.VMEM((_KPT, _G), jnp.int32),
          pltpu.VMEM_SHARED((_N, dp), jnp.float32),
          [pltpu.VMEM((_G, dp), jnp.float32) for _ in range(_NBUF)],
          [pltpu.SemaphoreType.DMA for _ in range(_NBUF)],
      ],
  )
  def sc_agg(p_hbm, src_hbm, dst_hbm, zeros_hbm, out_hbm,
             src_v, dst_v, agg_sh, rows, sems):
    c = lax.axis_index("c")
    s = lax.axis_index("s")
    w = c * _NSUB + s

    zbase = s * _ROWS_PER_SUB
    pltpu.sync_copy(zeros_hbm.at[pl.ds(zbase, _ROWS_PER_SUB)],
                    agg_sh.at[pl.ds(zbase, _ROWS_PER_SUB)])
    pltpu.sync_copy(src_hbm.at[pl.ds(w * _KPT, _KPT)], src_v)
    pltpu.sync_copy(dst_hbm.at[pl.ds(w * _KPT, _KPT)], dst_v)
    plsc.subcore_barrier()

    def body(q, carry):
      k0 = q * _NBUF
      descs = [
          pltpu.async_copy(p_hbm.at[src_v.at[k0 + j]], rows[j], sems[j])
          for j in range(_NBUF)
      ]
      for j in range(_NBUF):
        descs[j].wait()
        pltpu.sync_copy(rows[j], agg_sh.at[dst_v.at[k0 + j]], add=True)
      return carry

    lax.fori_loop(0, _KPT // _NBUF, body, 0)
    plsc.subcore_barrier()

    pltpu.sync_copy(agg_sh.at[pl.ds(zbase, _ROWS_PER_SUB)],
                    out_hbm.at[pl.ds(c * _N + zbase, _ROWS_PER_SUB)])

  return sc_agg


_sc_agg_128 = _make_sc_agg(_D)
_sc_agg_64 = _make_sc_agg(64)


def _tc_first(x, w_self, w_neigh, b):
  def body(x_ref, ws_ref, wn_ref, b_ref, s_ref, p_ref):
    xb = x_ref[...]
    s_ref[...] = jnp.dot(xb, ws_ref[...],
                         preferred_element_type=jnp.float32) + b_ref[...]
    p_ref[...] = jnp.dot(xb, wn_ref[...], preferred_element_type=jnp.float32)

  return pl.pallas_call(
      body,
      grid=(_N // _BN,),
      in_specs=[
          pl.BlockSpec((_BN, _D), lambda i: (i, 0)),
          pl.BlockSpec((_D, _D), lambda i: (0, 0)),
          pl.BlockSpec((_D, _D), lambda i: (0, 0)),
          pl.BlockSpec((1, _D), lambda i: (0, 0)),
      ],
      out_specs=[
          pl.BlockSpec((_BN, _D), lambda i: (i, 0)),
          pl.BlockSpec((_BN, _D), lambda i: (i, 0)),
      ],
      out_shape=[jax.ShapeDtypeStruct((_N, _D), jnp.float32)] * 2,
  )(x, w_self, w_neigh, b.reshape(1, _D))


def _tc_combine_next(s_prev, parts, deg, w_self, w_neigh, b, dp_in, dp_out):
  def body(s_ref, a0_ref, a1_ref, deg_ref, ws_ref, wn_ref, b_ref,
           s_out, p_out):
    h = s_ref[...] + (a0_ref[...] + a1_ref[...]) / deg_ref[...]
    h = jnp.maximum(h, 0.0)
    s_out[...] = jnp.dot(h, ws_ref[...],
                         preferred_element_type=jnp.float32) + b_ref[...]
    p_out[...] = jnp.dot(h, wn_ref[...], preferred_element_type=jnp.float32)

  nblk = _N // _BN
  return pl.pallas_call(
      body,
      grid=(nblk,),
      in_specs=[
          pl.BlockSpec((_BN, dp_in), lambda i: (i, 0)),
          pl.BlockSpec((_BN, dp_in), lambda i: (i, 0)),
          pl.BlockSpec((_BN, dp_in), lambda i: (i + nblk, 0)),
          pl.BlockSpec((_BN, 1), lambda i: (i, 0)),
          pl.BlockSpec((dp_in, dp_out), lambda i: (0, 0)),
          pl.BlockSpec((dp_in, dp_out), lambda i: (0, 0)),
          pl.BlockSpec((1, dp_out), lambda i: (0, 0)),
      ],
      out_specs=[
          pl.BlockSpec((_BN, dp_out), lambda i: (i, 0)),
          pl.BlockSpec((_BN, dp_out), lambda i: (i, 0)),
      ],
      out_shape=[jax.ShapeDtypeStruct((_N, dp_out), jnp.float32)] * 2,
  )(s_prev, parts, parts, deg, w_self, w_neigh, b.reshape(1, dp_out))


def _tc_final(s_prev, parts, deg, dp):
  def body(s_ref, a0_ref, a1_ref, deg_ref, o_ref):
    o_ref[...] = s_ref[...] + (a0_ref[...] + a1_ref[...]) / deg_ref[...]

  nblk = _N // _BN
  return pl.pallas_call(
      body,
      grid=(nblk,),
      in_specs=[
          pl.BlockSpec((_BN, dp), lambda i: (i, 0)),
          pl.BlockSpec((_BN, dp), lambda i: (i, 0)),
          pl.BlockSpec((_BN, dp), lambda i: (i + nblk, 0)),
          pl.BlockSpec((_BN, 1), lambda i: (i, 0)),
      ],
      out_specs=pl.BlockSpec((_BN, dp), lambda i: (i, 0)),
      out_shape=jax.ShapeDtypeStruct((_N, dp), jnp.float32),
  )(s_prev, parts, parts, deg)


def kernel(x, edge_index, in_degrees,
           W_self0, W_neigh0, b0,
           W_self1, W_neigh1, b1,
           W_self2, W_neigh2, b2):
  src = edge_index[0].reshape(_E // _G, _G)
  dst = edge_index[1].reshape(_E // _G, _G)
  deg = in_degrees.reshape(_N, 1)
  zeros128 = jnp.zeros((_N, _D), jnp.float32)
  zeros64 = jnp.zeros((_N, 64), jnp.float32)

  w_self2p = jnp.pad(W_self2, ((0, 0), (0, 64 - 47)))
  w_neigh2p = jnp.pad(W_neigh2, ((0, 0), (0, 64 - 47)))
  b2p = jnp.pad(b2, (0, 64 - 47))

  s0, p0 = _tc_first(x, W_self0, W_neigh0, b0)
  a0 = _sc_agg_128(p0, src, dst, zeros128)
  s1, p1 = _tc_combine_next(s0, a0, deg, W_self1, W_neigh1, b1, _D, _D)
  a1 = _sc_agg_128(p1, src, dst, zeros128)
  s2, p2 = _tc_combine_next(s1, a1, deg, w_self2p, w_neigh2p, b2p, _D, 64)
  a2 = _sc_agg_64(p2, src, dst, zeros64)
  out = _tc_final(s2, a2, deg, 64)
  return out[:, :47]


# speedup vs baseline: 11.7032x; 1.2657x over previous
import functools

import jax
import jax.numpy as jnp
from jax import lax
from jax.experimental import pallas as pl
from jax.experimental.pallas import tpu as pltpu
from jax.experimental.pallas import tpu_sc as plsc

_N = 10000
_E = 320000
_D = 128
_NCORES = 2
_NSUB = 16
_NW = _NCORES * _NSUB
_EW = _E // _NW
_G = 50
_KPT = _EW // _G
_NBUF = 4
_ROWS_PER_SUB = _N // _NSUB
_BN = 1000


def _make_sc_agg(dp: int):
  mesh = plsc.VectorSubcoreMesh(core_axis_name="c", subcore_axis_name="s")

  @functools.partial(
      pl.kernel,
      mesh=mesh,
      compiler_params=pltpu.CompilerParams(use_tc_tiling_on_sc=False),
      out_type=jax.ShapeDtypeStruct((_NCORES * _N, dp), jnp.float32),
      scratch_types=[
          pltpu.VMEM((_KPT, _G), jnp.int32),
          pltpu.VMEM((_KPT, _G), jnp.int32),
          pltpu.VMEM_SHARED((_N, dp), jnp.float32),
          [pltpu.VMEM((_G, dp), jnp.float32) for _ in range(_NBUF)],
          [pltpu.SemaphoreType.DMA for _ in range(_NBUF)],
          [pltpu.SemaphoreType.DMA for _ in range(_NBUF)],
      ],
  )
  def sc_agg(p_hbm, src_hbm, dst_hbm, zeros_hbm, out_hbm,
             src_v, dst_v, agg_sh, rows, sems_g, sems_s):
    c = lax.axis_index("c")
    s = lax.axis_index("s")
    w = c * _NSUB + s

    zbase = s * _ROWS_PER_SUB
    pltpu.sync_copy(zeros_hbm.at[pl.ds(zbase, _ROWS_PER_SUB)],
                    agg_sh.at[pl.ds(zbase, _ROWS_PER_SUB)])
    pltpu.sync_copy(src_hbm.at[pl.ds(w * _KPT, _KPT)], src_v)
    pltpu.sync_copy(dst_hbm.at[pl.ds(w * _KPT, _KPT)], dst_v)
    plsc.subcore_barrier()

    def body(q, carry):
      k0 = q * _NBUF
      descs = []
      for j in range(_NBUF):
        @pl.when(q > 0)
        def _(j=j):
          pltpu.make_async_copy(rows[j], agg_sh.at[dst_v.at[k0 + j]],
                                sems_s[j]).wait()
        descs.append(
            pltpu.async_copy(p_hbm.at[src_v.at[k0 + j]], rows[j], sems_g[j]))
      for j in range(_NBUF):
        descs[j].wait()
        pltpu.async_copy(rows[j], agg_sh.at[dst_v.at[k0 + j]], sems_s[j],
                         add=True)
      return carry

    lax.fori_loop(0, _KPT // _NBUF, body, 0)
    for j in range(_NBUF):
      pltpu.make_async_copy(rows[j], agg_sh.at[dst_v.at[j]],
                            sems_s[j]).wait()
    plsc.subcore_barrier()

    pltpu.sync_copy(agg_sh.at[pl.ds(zbase, _ROWS_PER_SUB)],
                    out_hbm.at[pl.ds(c * _N + zbase, _ROWS_PER_SUB)])

  return sc_agg


_sc_agg_128 = _make_sc_agg(_D)
_sc_agg_64 = _make_sc_agg(64)


def _tc_first(x, w_self, w_neigh, b):
  def body(x_ref, ws_ref, wn_ref, b_ref, s_ref, p_ref):
    xb = x_ref[...]
    s_ref[...] = jnp.dot(xb, ws_ref[...],
                         preferred_element_type=jnp.float32) + b_ref[...]
    p_ref[...] = jnp.dot(xb, wn_ref[...], preferred_element_type=jnp.float32)

  return pl.pallas_call(
      body,
      grid=(_N // _BN,),
      in_specs=[
          pl.BlockSpec((_BN, _D), lambda i: (i, 0)),
          pl.BlockSpec((_D, _D), lambda i: (0, 0)),
          pl.BlockSpec((_D, _D), lambda i: (0, 0)),
          pl.BlockSpec((1, _D), lambda i: (0, 0)),
      ],
      out_specs=[
          pl.BlockSpec((_BN, _D), lambda i: (i, 0)),
          pl.BlockSpec((_BN, _D), lambda i: (i, 0)),
      ],
      out_shape=[jax.ShapeDtypeStruct((_N, _D), jnp.float32)] * 2,
  )(x, w_self, w_neigh, b.reshape(1, _D))


def _tc_combine_next(s_prev, parts, deg, w_self, w_neigh, b, dp_in, dp_out):
  def body(s_ref, a0_ref, a1_ref, deg_ref, ws_ref, wn_ref, b_ref,
           s_out, p_out):
    h = s_ref[...] + (a0_ref[...] + a1_ref[...]) / deg_ref[...]
    h = jnp.maximum(h, 0.0)
    s_out[...] = jnp.dot(h, ws_ref[...],
                         preferred_element_type=jnp.float32) + b_ref[...]
    p_out[...] = jnp.dot(h, wn_ref[...], preferred_element_type=jnp.float32)

  nblk = _N // _BN
  return pl.pallas_call(
      body,
      grid=(nblk,),
      in_specs=[
          pl.BlockSpec((_BN, dp_in), lambda i: (i, 0)),
          pl.BlockSpec((_BN, dp_in), lambda i: (i, 0)),
          pl.BlockSpec((_BN, dp_in), lambda i: (i + nblk, 0)),
          pl.BlockSpec((_BN, 1), lambda i: (i, 0)),
          pl.BlockSpec((dp_in, dp_out), lambda i: (0, 0)),
          pl.BlockSpec((dp_in, dp_out), lambda i: (0, 0)),
          pl.BlockSpec((1, dp_out), lambda i: (0, 0)),
      ],
      out_specs=[
          pl.BlockSpec((_BN, dp_out), lambda i: (i, 0)),
          pl.BlockSpec((_BN, dp_out), lambda i: (i, 0)),
      ],
      out_shape=[jax.ShapeDtypeStruct((_N, dp_out), jnp.float32)] * 2,
  )(s_prev, parts, parts, deg, w_self, w_neigh, b.reshape(1, dp_out))


def _tc_final(s_prev, parts, deg, dp):
  def body(s_ref, a0_ref, a1_ref, deg_ref, o_ref):
    o_ref[...] = s_ref[...] + (a0_ref[...] + a1_ref[...]) / deg_ref[...]

  nblk = _N // _BN
  return pl.pallas_call(
      body,
      grid=(nblk,),
      in_specs=[
          pl.BlockSpec((_BN, dp), lambda i: (i, 0)),
          pl.BlockSpec((_BN, dp), lambda i: (i, 0)),
          pl.BlockSpec((_BN, dp), lambda i: (i + nblk, 0)),
          pl.BlockSpec((_BN, 1), lambda i: (i, 0)),
      ],
      out_specs=pl.BlockSpec((_BN, dp), lambda i: (i, 0)),
      out_shape=jax.ShapeDtypeStruct((_N, dp), jnp.float32),
  )(s_prev, parts, parts, deg)


def kernel(x, edge_index, in_degrees,
           W_self0, W_neigh0, b0,
           W_self1, W_neigh1, b1,
           W_self2, W_neigh2, b2):
  src = edge_index[0].reshape(_E // _G, _G)
  dst = edge_index[1].reshape(_E // _G, _G)
  deg = in_degrees.reshape(_N, 1)
  zeros128 = jnp.zeros((_N, _D), jnp.float32)
  zeros64 = jnp.zeros((_N, 64), jnp.float32)

  w_self2p = jnp.pad(W_self2, ((0, 0), (0, 64 - 47)))
  w_neigh2p = jnp.pad(W_neigh2, ((0, 0), (0, 64 - 47)))
  b2p = jnp.pad(b2, (0, 64 - 47))

  s0, p0 = _tc_first(x, W_self0, W_neigh0, b0)
  a0 = _sc_agg_128(p0, src, dst, zeros128)
  s1, p1 = _tc_combine_next(s0, a0, deg, W_self1, W_neigh1, b1, _D, _D)
  a1 = _sc_agg_128(p1, src, dst, zeros128)
  s2, p2 = _tc_combine_next(s1, a1, deg, w_self2p, w_neigh2p, b2p, _D, 64)
  a2 = _sc_agg_64(p2, src, dst, zeros64)
  out = _tc_final(s2, a2, deg, 64)
  return out[:, :47]
